# baseline (speedup 1.0000x reference)
import jax
import jax.numpy as jnp
from jax import lax
from jax.experimental import pallas as pl
from jax.experimental.pallas import tpu as pltpu
from jax.experimental.pallas import tpu_sc as plsc

_B, _S, _D = 4, 2048, 512
_NC, _NS, _L = 2, 16, 16
_NW = _NC * _NS
_N = _B * _S
_RPW = _N // _NW
_C = 32
_NCHUNK = _RPW // _C


def _emb_body(x_hbm, table_hbm, pe_hbm, out_hbm, idx_v, rows_v, pe_v, sem):
    wid = lax.axis_index("s") * _NC + lax.axis_index("c")
    base = wid * _RPW
    s_base = base % _S
    pltpu.sync_copy(x_hbm.at[pl.ds(base, _RPW)], idx_v)

    @pl.loop(0, _NCHUNK)
    def _chunk(i):
        off = i * _C
        gather = pltpu.async_copy(
            table_hbm.at[idx_v.at[pl.ds(off, _C)]], rows_v, sem)
        pltpu.sync_copy(pe_hbm.at[pl.ds(s_base + off, _C)], pe_v)
        gather.wait()

        @pl.loop(0, _C)
        def _row(r):
            for c in range(_D // _L):
                sl = pl.ds(c * _L, _L)
                rows_v[r, sl] += pe_v[r, sl]

        pltpu.sync_copy(rows_v, out_hbm.at[pl.ds(base + off, _C)])


def kernel(x, table, pe):
    xf = x.reshape(-1).astype(jnp.int32)
    mesh = plsc.VectorSubcoreMesh(core_axis_name="c", subcore_axis_name="s")
    out = pl.kernel(
        _emb_body,
        out_type=jax.ShapeDtypeStruct((_N, _D), jnp.float32),
        mesh=mesh,
        scratch_types=[
            pltpu.VMEM((_RPW,), jnp.int32),
            pltpu.VMEM((_C, _D), jnp.float32),
            pltpu.VMEM((_C, _D), jnp.float32),
            pltpu.SemaphoreType.DMA,
        ],
    )(xf, table, pe)
    return out.reshape(_B, _S, _D)


# speedup vs baseline: 1.2044x; 1.2044x over previous
import jax
import jax.numpy as jnp
from jax import lax
from jax.experimental import pallas as pl
from jax.experimental.pallas import tpu as pltpu
from jax.experimental.pallas import tpu_sc as plsc

_B, _S, _D = 4, 2048, 512
_NC, _NS, _L = 2, 16, 16
_NW = _NC * _NS
_N = _B * _S
_RPW = _N // _NW
_C = 32
_NCHUNK = _RPW // _C
_NB = 3


def _emb_body(x_hbm, table_hbm, pe_hbm, out_hbm,
              idx_v, rows_v, pe_v, gsem, psem, ssem):
    wid = lax.axis_index("s") * _NC + lax.axis_index("c")
    base = wid * _RPW
    s_base = base % _S
    pltpu.sync_copy(x_hbm.at[pl.ds(base, _RPW)], idx_v)

    def start_fetch(i):
        b = i % _NB
        pltpu.async_copy(table_hbm.at[idx_v.at[pl.ds(i * _C, _C)]],
                         rows_v.at[b], gsem.at[b])
        pltpu.async_copy(pe_hbm.at[pl.ds(s_base + i * _C, _C)],
                         pe_v.at[b], psem.at[b])

    def wait_fetch(i):
        b = i % _NB
        pltpu.make_async_copy(table_hbm.at[idx_v.at[pl.ds(i * _C, _C)]],
                              rows_v.at[b], gsem.at[b]).wait()
        pltpu.make_async_copy(pe_hbm.at[pl.ds(s_base + i * _C, _C)],
                              pe_v.at[b], psem.at[b]).wait()

    for i in range(_NB):
        start_fetch(i)

    for i in range(_NCHUNK):
        b = i % _NB
        wait_fetch(i)

        rv = rows_v.at[b]
        pv = pe_v.at[b]

        @pl.loop(0, _C)
        def _row(r):
            for c in range(_D // _L):
                sl = pl.ds(c * _L, _L)
                rv[r, sl] += pv[r, sl]

        out_slice = out_hbm.at[pl.ds(base + i * _C, _C)]
        pltpu.async_copy(rows_v.at[b], out_slice, ssem.at[b])
        if i + _NB < _NCHUNK:
            pltpu.make_async_copy(rows_v.at[b], out_slice, ssem.at[b]).wait()
            start_fetch(i + _NB)

    for i in range(_NCHUNK - _NB, _NCHUNK):
        b = i % _NB
        pltpu.make_async_copy(rows_v.at[b],
                              out_hbm.at[pl.ds(base + i * _C, _C)],
                              ssem.at[b]).wait()


def kernel(x, table, pe):
    xf = x.reshape(-1).astype(jnp.int32)
    mesh = plsc.VectorSubcoreMesh(core_axis_name="c", subcore_axis_name="s")
    out = pl.kernel(
        _emb_body,
        out_type=jax.ShapeDtypeStruct((_N, _D), jnp.float32),
        mesh=mesh,
        scratch_types=[
            pltpu.VMEM((_RPW,), jnp.int32),
            pltpu.VMEM((_NB, _C, _D), jnp.float32),
            pltpu.VMEM((_NB, _C, _D), jnp.float32),
            pltpu.SemaphoreType.DMA((_NB,)),
            pltpu.SemaphoreType.DMA((_NB,)),
            pltpu.SemaphoreType.DMA((_NB,)),
        ],
    )(xf, table, pe)
    return out.reshape(_B, _S, _D)


# speedup vs baseline: 1.2058x; 1.0012x over previous
import jax
import jax.numpy as jnp
from jax import lax
from jax.experimental import pallas as pl
from jax.experimental.pallas import tpu as pltpu
from jax.experimental.pallas import tpu_sc as plsc

_B, _S, _D = 4, 2048, 512
_NC, _NS, _L = 2, 16, 16
_NW = _NC * _NS
_N = _B * _S
_SW = _S // _NW
_C = 32
_NCHUNK = (_B * _SW) // _C
_NB = 4


def _emb_body(x_hbm, table_hbm, pe_hbm, out_hbm,
              idx_v, pe_v, rows_v, isem, psem, gsem, ssem):
    wid = lax.axis_index("s") * _NC + lax.axis_index("c")
    s0 = wid * _SW

    pltpu.async_copy(pe_hbm.at[pl.ds(s0, _SW)], pe_v, psem)
    for b in range(_B):
        pltpu.async_copy(x_hbm.at[pl.ds(b * _S + s0, _SW)], idx_v.at[b], isem)
    for b in range(_B):
        pltpu.make_async_copy(x_hbm.at[pl.ds(b * _S + s0, _SW)],
                              idx_v.at[b], isem).wait()

    def chunk_coords(i):
        b, h = divmod(i, _SW // _C)
        return b, h

    def start_gather(i):
        b, h = chunk_coords(i)
        pltpu.async_copy(table_hbm.at[idx_v.at[b, pl.ds(h * _C, _C)]],
                         rows_v.at[i % _NB], gsem.at[i % _NB])

    def out_slice(i):
        b, h = chunk_coords(i)
        return out_hbm.at[pl.ds(b * _S + s0 + h * _C, _C)]

    for i in range(_NB):
        start_gather(i)

    pltpu.make_async_copy(pe_hbm.at[pl.ds(s0, _SW)], pe_v, psem).wait()

    for i in range(_NCHUNK):
        b, h = chunk_coords(i)
        r0 = i % _NB
        pltpu.make_async_copy(table_hbm.at[idx_v.at[b, pl.ds(h * _C, _C)]],
                              rows_v.at[r0], gsem.at[r0]).wait()
        rv = rows_v.at[r0]

        @pl.loop(0, _C)
        def _row(r):
            for c in range(_D // _L):
                sl = pl.ds(c * _L, _L)
                rv[r, sl] += pe_v[h * _C + r, sl]

        pltpu.async_copy(rv, out_slice(i), ssem.at[r0])
        if i + _NB < _NCHUNK:
            pltpu.make_async_copy(rv, out_slice(i), ssem.at[r0]).wait()
            start_gather(i + _NB)

    for i in range(_NCHUNK - _NB, _NCHUNK):
        pltpu.make_async_copy(rows_v.at[i % _NB], out_slice(i),
                              ssem.at[i % _NB]).wait()


def kernel(x, table, pe):
    mesh = plsc.VectorSubcoreMesh(core_axis_name="c", subcore_axis_name="s")
    out = pl.kernel(
        _emb_body,
        out_type=jax.ShapeDtypeStruct((_N, _D), jnp.float32),
        mesh=mesh,
        scratch_types=[
            pltpu.VMEM((_B, _SW), jnp.int32),
            pltpu.VMEM((_SW, _D), jnp.float32),
            pltpu.VMEM((_NB, _C, _D), jnp.float32),
            pltpu.SemaphoreType.DMA,
            pltpu.SemaphoreType.DMA,
            pltpu.SemaphoreType.DMA((_NB,)),
            pltpu.SemaphoreType.DMA((_NB,)),
        ],
    )(x.reshape(-1).astype(jnp.int32), table, pe)
    return out.reshape(_B, _S, _D)
